# baseline (speedup 1.0000x reference)
import functools
import math

import jax
import jax.numpy as jnp
from jax import lax
from jax.experimental import pallas as pl
from jax.experimental.pallas import tpu as pltpu
from jax.experimental.pallas import tpu_sc as plsc

N = 10000
E = 320000
F_IN = 128
H = 64
C = 2
L = 3
K = int(math.ceil(0.25 * N))

NP = 10240
NC = 2
NS = 16
NW = NC * NS
RPT = NP // NS
EPW = E // NW
EPT = E // NS
CH = 400
NCH = EPW // CH

@functools.cache
def _get_spmv():
    mesh = plsc.VectorSubcoreMesh(core_axis_name="c", subcore_axis_name="s")
    return functools.partial(
        pl.kernel,
        mesh=mesh,
        out_type=jax.ShapeDtypeStruct((NC, NP, H), jnp.float32),
        scratch_types=[
            pltpu.VMEM_SHARED((NP, H), jnp.float32),
            pltpu.VMEM((CH, H), jnp.float32),
            pltpu.VMEM((CH, H), jnp.float32),
            pltpu.VMEM((NCH, CH), jnp.int32),
            pltpu.VMEM((NCH, CH), jnp.int32),
            pltpu.VMEM((NCH, CH), jnp.float32),
            pltpu.SemaphoreType.DMA,
            pltpu.SemaphoreType.DMA,
        ],
        compiler_params=pltpu.CompilerParams(needs_layout_passes=False, use_tc_tiling_on_sc=False),
    )(_sc_spmv_body)


def _sc_spmv_body(ya, src_h, dst_h, ew_h, zer_h, out, acc_s, gbA, gbB, srcb,
                  dstb, ewb, semA, semB):
    cid = lax.axis_index("c")
    tid = lax.axis_index("s")
    wid = cid * NS + tid
    rbase = tid * RPT
    cbase = wid * NCH

    pltpu.sync_copy(zer_h.at[pl.ds(rbase, RPT)], acc_s.at[pl.ds(rbase, RPT)])
    pltpu.sync_copy(src_h.at[pl.ds(cbase, NCH)], srcb)
    pltpu.sync_copy(dst_h.at[pl.ds(cbase, NCH)], dstb)
    pltpu.sync_copy(ew_h.at[pl.ds(cbase, NCH)], ewb)
    plsc.subcore_barrier()

    def _gather_start(j, gb, sem):
        pltpu.async_copy(ya.at[srcb.at[j]], gb, sem)

    def _process(j, gb, sem):
        pltpu.make_async_copy(ya.at[srcb.at[j]], gb, sem).wait()

        def _scale(g, _):
            ewv = ewb[j, pl.ds(g * 16, 16)]
            for b in range(16):
                w = jnp.full((16,), ewv[b])
                e = g * 16 + b
                for q in range(H // 16):
                    gb[e, pl.ds(q * 16, 16)] = gb[e, pl.ds(q * 16, 16)] * w
            return 0
        lax.fori_loop(0, CH // 16, _scale, 0)
        pltpu.sync_copy(gb, acc_s.at[dstb.at[j]], add=True)

    _gather_start(0, gbA, semA)

    def _pipe(i, _):
        j = i * 2
        _gather_start(j + 1, gbB, semB)
        _process(j, gbA, semA)
        _gather_start(j + 2, gbA, semA)
        _process(j + 1, gbB, semB)
        return 0

    lax.fori_loop(0, (NCH - 1) // 2, _pipe, 0)
    _process(NCH - 1, gbA, semA)
    plsc.subcore_barrier()
    pltpu.sync_copy(acc_s.at[pl.ds(rbase, RPT)], out.at[cid, pl.ds(rbase, RPT)])


@functools.cache
def _get_vchain():
    mesh = plsc.VectorSubcoreMesh(core_axis_name="c", subcore_axis_name="s")
    return functools.partial(
        pl.kernel,
        mesh=mesh,
        out_type=jax.ShapeDtypeStruct((NP,), jnp.float32),
        scratch_types=[
            pltpu.VMEM_SHARED((NP,), jnp.float32),
            pltpu.VMEM((NP,), jnp.float32),
            pltpu.VMEM((RPT,), jnp.float32),
            pltpu.VMEM((EPT,), jnp.int32),
            pltpu.VMEM((EPT,), jnp.int32),
            pltpu.VMEM((EPT,), jnp.float32),
            pltpu.VMEM((EPT,), jnp.float32),
            pltpu.VMEM((L + 1, 16), jnp.float32),
        ],
        compiler_params=pltpu.CompilerParams(needs_layout_passes=False, use_tc_tiling_on_sc=False),
    )(_sc_vchain_body)


def _sc_vchain_body(y0_h, src_h, dst_h, ew_h, zer_h, pw_h, out, acc_s, ytile,
                    res_t, srcb, dstb, ewb, vbuf, wbuf):
    cid = lax.axis_index("c")
    tid = lax.axis_index("s")

    @pl.when(cid == 0)
    def _():
        rbase = tid * RPT
        ebase = tid * EPT
        pltpu.sync_copy(src_h.at[pl.ds(ebase, EPT)], srcb)
        pltpu.sync_copy(dst_h.at[pl.ds(ebase, EPT)], dstb)
        pltpu.sync_copy(ew_h.at[pl.ds(ebase, EPT)], ewb)
        pltpu.sync_copy(pw_h, wbuf)
        pltpu.sync_copy(y0_h, ytile)
        pltpu.sync_copy(zer_h.at[pl.ds(rbase, RPT)], acc_s.at[pl.ds(rbase, RPT)])

        w0 = wbuf[0]

        def _init(i, _):
            res_t[pl.ds(i * 16, 16)] = ytile[pl.ds(rbase + i * 16, 16)] * w0
            return 0
        lax.fori_loop(0, RPT // 16, _init, 0)
        plsc.subcore_barrier()

        for k in range(1, L + 1):
            def _edges(i, _):
                sv = srcb[pl.ds(i * 16, 16)]
                vals = plsc.load_gather(ytile, [sv])
                vbuf[pl.ds(i * 16, 16)] = vals * ewb[pl.ds(i * 16, 16)]
                return 0
            lax.fori_loop(0, EPT // 16, _edges, 0)
            pltpu.sync_copy(vbuf, acc_s.at[dstb], add=True)
            plsc.subcore_barrier()
            pltpu.sync_copy(acc_s, ytile)
            plsc.subcore_barrier()
            if k < L:
                pltpu.sync_copy(zer_h.at[pl.ds(rbase, RPT)],
                                acc_s.at[pl.ds(rbase, RPT)])
            wk = wbuf[k]

            def _accum(i, _):
                res_t[pl.ds(i * 16, 16)] = (res_t[pl.ds(i * 16, 16)]
                                            + ytile[pl.ds(rbase + i * 16, 16)] * wk)
                return 0
            lax.fori_loop(0, RPT // 16, _accum, 0)
            plsc.subcore_barrier()

        pltpu.sync_copy(res_t, out.at[pl.ds(rbase, RPT)])


def _tc_call(body, out_shapes, *args):
    return pl.pallas_call(
        body,
        out_shape=out_shapes,
    )(*args)


def _k_xw1(x_ref, w_ref, o_ref):
    o_ref[...] = jnp.dot(x_ref[...], w_ref[...],
                         preferred_element_type=jnp.float32)


def _k_dis_um(deg_ref, u0_ref, dis_ref, um_ref):
    deg = deg_ref[...]
    pos = deg > 0.0
    dis = jnp.where(pos, lax.rsqrt(jnp.where(pos, deg, 1.0)), 0.0)
    dis_ref[...] = dis
    um_ref[...] = dis[:, None] * u0_ref[...]


def _topk_mask(score2d):
    ibits = lax.bitcast_convert_type(score2d, jnp.int32)
    key = jnp.where(ibits >= 0, ibits, ibits ^ jnp.int32(0x7FFFFFFF))
    u = key ^ jnp.int32(-2147483648)
    hi = lax.shift_right_logical(u, 16)
    lo = u & jnp.int32(0xFFFF)

    def _find(vals, base_cnt, extra):
        def body(j, t):
            cand = t + lax.shift_left(jnp.int32(1), jnp.int32(15) - j)
            cnt = base_cnt + jnp.sum(jnp.where(extra & (vals >= cand), 1, 0))
            return jnp.where(cnt >= K, cand, t)
        return lax.fori_loop(0, 16, body, jnp.int32(0))

    ones = jnp.ones_like(hi, dtype=jnp.bool_)
    th = _find(hi, jnp.int32(0), ones)
    cnt_hi_gt = jnp.sum(jnp.where(hi > th, 1, 0))
    eq_hi = hi == th
    tl = _find(lo, cnt_hi_gt, eq_hi)
    gt = (hi > th) | (eq_hi & (lo > tl))
    eq = eq_hi & (lo == tl)
    r = K - jnp.sum(jnp.where(gt, 1, 0))

    eqf = jnp.where(eq, 1.0, 0.0)
    col = lax.broadcasted_iota(jnp.int32, (128, 128), 0)
    row = lax.broadcasted_iota(jnp.int32, (128, 128), 1)
    ut = jnp.where(col <= row, 1.0, 0.0)
    within = jnp.dot(eqf, ut, preferred_element_type=jnp.float32)
    rowtot = within[:, 127:128]
    a = lax.broadcasted_iota(jnp.int32, (80, 80), 0)
    b = lax.broadcasted_iota(jnp.int32, (80, 80), 1)
    sl = jnp.where(b < a, 1.0, 0.0)
    before = jnp.dot(sl, rowtot, preferred_element_type=jnp.float32)
    cum = before + within
    sel_eq = eq & (cum <= r.astype(jnp.float32))
    return jnp.where(gt | sel_eq, 1.0, 0.0)


def _k_add(p_ref, o_ref):
    o_ref[...] = p_ref[0] + p_ref[1]


def _k_score(um_ref, t1_ref, t2_ref, t3_ref, dis_ref, s2_ref, b1_ref, p_ref,
             beta_ref, pw_ref, h_ref, score_ref):
    pw = pw_ref[...]
    dis = dis_ref[...]
    m = (pw[0] * um_ref[...]
         + pw[1] * t1_ref[...]
         + pw[2] * t2_ref[...]
         + pw[3] * t3_ref[...])
    h = jnp.maximum(dis[:, None] * m + b1_ref[...][None, :], 0.0)
    rowid = lax.broadcasted_iota(jnp.int32, (NP, 1), 0)[:, 0]
    valid = rowid < N
    h = jnp.where(valid[:, None], h, 0.0)
    score1 = jnp.dot(h, p_ref[...][:, None],
                     preferred_element_type=jnp.float32)[:, 0]
    beta = beta_ref[...]
    score = jnp.tanh(beta[0] * score1 + beta[1] * (dis * s2_ref[...]))
    score = jnp.where(valid, score, -3.0e38)
    h_ref[...] = h
    score_ref[...] = score


def _k_mask(score2d_ref, mask2d_ref):
    mask2d_ref[...] = _topk_mask(score2d_ref[...])


def _k_xw(h_ref, score_ref, mask_ref, dis_ref, w2_ref, xw_ref, v0_ref):
    maskf = mask_ref[...]
    score = score_ref[...]
    xp = maskf[:, None] * h_ref[...] * score[:, None]
    xw_ref[...] = jnp.dot(xp, w2_ref[...], preferred_element_type=jnp.float32)
    v0_ref[...] = dis_ref[...] * maskf


def _k_deg2(mch_ref, dis_ref, mask_ref, xw_ref, d2_ref, z_ref, u2_ref):
    maskf = mask_ref[...]
    dis = dis_ref[...]
    deg2 = maskf * (dis * mch_ref[...]) + maskf
    pos = deg2 > 0.0
    d2 = jnp.where(pos, lax.rsqrt(jnp.where(pos, deg2, 1.0)), 0.0) * maskf
    z = d2[:, None] * xw_ref[...]
    d2_ref[...] = d2
    z_ref[...] = z
    u2_ref[...] = dis[:, None] * z


def _k_final(u2_ref, s1_ref, s2_ref, s3_ref, dis_ref, mask_ref, d2_ref,
             z_ref, b2_ref, w3_ref, b3_ref, pw_ref, out_ref):
    pw = pw_ref[...]
    m2 = (pw[0] * u2_ref[...]
          + pw[1] * s1_ref[...]
          + pw[2] * s2_ref[...]
          + pw[3] * s3_ref[...])
    dis = dis_ref[...]
    maskf = mask_ref[...]
    d2 = d2_ref[...]
    a2z = maskf[:, None] * (dis[:, None] * m2) + z_ref[...]
    h2 = maskf[:, None] * jnp.maximum(d2[:, None] * a2z + b2_ref[...][None, :], 0.0)
    g = jnp.sum(h2, axis=0, keepdims=True)
    logits = jnp.dot(g, w3_ref[...], preferred_element_type=jnp.float32) \
        + b3_ref[...][None, :]
    mx = jnp.max(logits, axis=-1, keepdims=True)
    sh = logits - mx
    out_ref[...] = sh - jnp.log(jnp.sum(jnp.exp(sh), axis=-1, keepdims=True))


def kernel(x, edge_index, edge_weight, batch, W1, b1, pan_w, p, beta, W2, b2,
           W3, b3):
    src = edge_index[0]
    dst = edge_index[1]
    ew = edge_weight
    src2 = src.reshape(E // CH, CH)
    dst2 = dst.reshape(E // CH, CH)
    ew2 = ew.reshape(E // CH, CH)
    xp = jnp.pad(x, ((0, NP - N), (0, 0)))
    zer_nh = jnp.zeros((NP, H), jnp.float32)
    zer_n = jnp.zeros((NP,), jnp.float32)
    ones_n = jnp.pad(jnp.ones((N,), jnp.float32), (0, NP - N))
    pw16 = jnp.broadcast_to(pan_w[:, None], (L + 1, 16))

    u0 = _tc_call(_k_xw1, jax.ShapeDtypeStruct((NP, H), jnp.float32), xp, W1)

    deg = _get_vchain()(ones_n, src, dst, ew, zer_n, pw16)
    dis, um = _tc_call(
        _k_dis_um,
        (jax.ShapeDtypeStruct((NP,), jnp.float32),
         jax.ShapeDtypeStruct((NP, H), jnp.float32)),
        deg, u0)

    s2raw = _get_vchain()(dis, dst, src, ew, zer_n, pw16)

    nh = jax.ShapeDtypeStruct((NP, H), jnp.float32)
    t1 = _tc_call(_k_add, nh, _get_spmv()(um, src2, dst2, ew2, zer_nh))
    t2 = _tc_call(_k_add, nh, _get_spmv()(t1, src2, dst2, ew2, zer_nh))
    t3 = _tc_call(_k_add, nh, _get_spmv()(t2, src2, dst2, ew2, zer_nh))

    h, score = _tc_call(
        _k_score,
        (jax.ShapeDtypeStruct((NP, H), jnp.float32),
         jax.ShapeDtypeStruct((NP,), jnp.float32)),
        um, t1, t2, t3, dis, s2raw, b1, p, beta, pan_w)
    mask2d = _tc_call(_k_mask, jax.ShapeDtypeStruct((80, 128), jnp.float32),
                      score.reshape(80, 128))
    maskf = mask2d.reshape(NP)
    xw, v0 = _tc_call(
        _k_xw,
        (jax.ShapeDtypeStruct((NP, H), jnp.float32),
         jax.ShapeDtypeStruct((NP,), jnp.float32)),
        h, score, maskf, dis, W2)

    mch = _get_vchain()(v0, src, dst, ew, zer_n, pw16)
    d2, z, u2 = _tc_call(
        _k_deg2,
        (jax.ShapeDtypeStruct((NP,), jnp.float32),
         jax.ShapeDtypeStruct((NP, H), jnp.float32),
         jax.ShapeDtypeStruct((NP, H), jnp.float32)),
        mch, dis, maskf, xw)

    c1 = _tc_call(_k_add, nh, _get_spmv()(u2, src2, dst2, ew2, zer_nh))
    c2 = _tc_call(_k_add, nh, _get_spmv()(c1, src2, dst2, ew2, zer_nh))
    c3 = _tc_call(_k_add, nh, _get_spmv()(c2, src2, dst2, ew2, zer_nh))

    out = _tc_call(
        _k_final, jax.ShapeDtypeStruct((1, C), jnp.float32),
        u2, c1, c2, c3, dis, maskf, d2, z, b2, W3, b3, pan_w)
    return out


# speedup vs baseline: 1.1406x; 1.1406x over previous
import functools
import math

import jax
import jax.numpy as jnp
from jax import lax
from jax.experimental import pallas as pl
from jax.experimental.pallas import tpu as pltpu
from jax.experimental.pallas import tpu_sc as plsc

N = 10000
E = 320000
F_IN = 128
H = 64
C = 2
L = 3
K = int(math.ceil(0.25 * N))

NP = 10240
NC = 2
NS = 16
NW = NC * NS
RPT = NP // NS
EPW = E // NW
EPT = E // NS
CH = 400
NCH = EPW // CH

@functools.cache
def _get_spmv():
    mesh = plsc.VectorSubcoreMesh(core_axis_name="c", subcore_axis_name="s")
    return functools.partial(
        pl.kernel,
        mesh=mesh,
        out_type=jax.ShapeDtypeStruct((NC, NP, H), jnp.float32),
        scratch_types=[
            pltpu.VMEM_SHARED((NP, H), jnp.float32),
            pltpu.VMEM_SHARED((NP, H), jnp.float32),
            pltpu.VMEM((CH, H), jnp.float32),
            pltpu.VMEM((CH,), jnp.int32),
            pltpu.VMEM((CH,), jnp.int32),
            pltpu.VMEM((CH,), jnp.float32),
            pltpu.SemaphoreType.DMA,
        ],
        compiler_params=pltpu.CompilerParams(needs_layout_passes=False, use_tc_tiling_on_sc=False),
    )(_sc_spmv_body)


def _sc_spmv_body(ya, src_h, dst_h, ew_h, zer_h, out, acc_s, y_s, gbuf,
                  srcb, dstb, ewb, sem):
    cid = lax.axis_index("c")
    tid = lax.axis_index("s")
    wid = cid * NS + tid
    rbase = tid * RPT
    cbase = wid * NCH

    pltpu.sync_copy(zer_h.at[pl.ds(rbase, RPT)], acc_s.at[pl.ds(rbase, RPT)])
    pltpu.sync_copy(ya.at[pl.ds(rbase, RPT)], y_s.at[pl.ds(rbase, RPT)])
    plsc.subcore_barrier()

    def _chunk(j, _):
        pltpu.sync_copy(src_h.at[cbase + j], srcb)
        pltpu.sync_copy(dst_h.at[cbase + j], dstb)
        pltpu.sync_copy(ew_h.at[cbase + j], ewb)
        pltpu.async_copy(y_s.at[srcb], gbuf, sem).wait()

        def _scale(g, _):
            ewv = ewb[pl.ds(g * 16, 16)]
            for b in range(16):
                w = jnp.full((16,), ewv[b])
                e = g * 16 + b
                for q in range(H // 16):
                    gbuf[e, pl.ds(q * 16, 16)] = gbuf[e, pl.ds(q * 16, 16)] * w
            return 0
        lax.fori_loop(0, CH // 16, _scale, 0)
        pltpu.sync_copy(gbuf, acc_s.at[dstb], add=True)
        return 0

    lax.fori_loop(0, NCH, _chunk, 0)
    plsc.subcore_barrier()
    pltpu.sync_copy(acc_s.at[pl.ds(rbase, RPT)], out.at[cid, pl.ds(rbase, RPT)])


@functools.cache
def _get_vchain():
    mesh = plsc.VectorSubcoreMesh(core_axis_name="c", subcore_axis_name="s")
    return functools.partial(
        pl.kernel,
        mesh=mesh,
        out_type=jax.ShapeDtypeStruct((NP,), jnp.float32),
        scratch_types=[
            pltpu.VMEM_SHARED((NP,), jnp.float32),
            pltpu.VMEM((NP,), jnp.float32),
            pltpu.VMEM((RPT,), jnp.float32),
            pltpu.VMEM((EPT,), jnp.int32),
            pltpu.VMEM((EPT,), jnp.int32),
            pltpu.VMEM((EPT,), jnp.float32),
            pltpu.VMEM((EPT,), jnp.float32),
            pltpu.VMEM((L + 1, 16), jnp.float32),
        ],
        compiler_params=pltpu.CompilerParams(needs_layout_passes=False, use_tc_tiling_on_sc=False),
    )(_sc_vchain_body)


def _sc_vchain_body(y0_h, src_h, dst_h, ew_h, zer_h, pw_h, out, acc_s, ytile,
                    res_t, srcb, dstb, ewb, vbuf, wbuf):
    cid = lax.axis_index("c")
    tid = lax.axis_index("s")

    @pl.when(cid == 0)
    def _():
        rbase = tid * RPT
        ebase = tid * EPT
        pltpu.sync_copy(src_h.at[pl.ds(ebase, EPT)], srcb)
        pltpu.sync_copy(dst_h.at[pl.ds(ebase, EPT)], dstb)
        pltpu.sync_copy(ew_h.at[pl.ds(ebase, EPT)], ewb)
        pltpu.sync_copy(pw_h, wbuf)
        pltpu.sync_copy(y0_h, ytile)
        pltpu.sync_copy(zer_h.at[pl.ds(rbase, RPT)], acc_s.at[pl.ds(rbase, RPT)])

        w0 = wbuf[0]

        def _init(i, _):
            res_t[pl.ds(i * 16, 16)] = ytile[pl.ds(rbase + i * 16, 16)] * w0
            return 0
        lax.fori_loop(0, RPT // 16, _init, 0)
        plsc.subcore_barrier()

        for k in range(1, L + 1):
            def _edges(i, _):
                sv = srcb[pl.ds(i * 16, 16)]
                vals = plsc.load_gather(ytile, [sv])
                vbuf[pl.ds(i * 16, 16)] = vals * ewb[pl.ds(i * 16, 16)]
                return 0
            lax.fori_loop(0, EPT // 16, _edges, 0)
            pltpu.sync_copy(vbuf, acc_s.at[dstb], add=True)
            plsc.subcore_barrier()
            pltpu.sync_copy(acc_s, ytile)
            plsc.subcore_barrier()
            if k < L:
                pltpu.sync_copy(zer_h.at[pl.ds(rbase, RPT)],
                                acc_s.at[pl.ds(rbase, RPT)])
            wk = wbuf[k]

            def _accum(i, _):
                res_t[pl.ds(i * 16, 16)] = (res_t[pl.ds(i * 16, 16)]
                                            + ytile[pl.ds(rbase + i * 16, 16)] * wk)
                return 0
            lax.fori_loop(0, RPT // 16, _accum, 0)
            plsc.subcore_barrier()

        pltpu.sync_copy(res_t, out.at[pl.ds(rbase, RPT)])


def _tc_call(body, out_shapes, *args):
    return pl.pallas_call(
        body,
        out_shape=out_shapes,
    )(*args)


def _k_xw1(x_ref, w_ref, o_ref):
    o_ref[...] = jnp.dot(x_ref[...], w_ref[...],
                         preferred_element_type=jnp.float32)


def _k_dis_um(deg_ref, u0_ref, dis_ref, um_ref):
    deg = deg_ref[...]
    pos = deg > 0.0
    dis = jnp.where(pos, lax.rsqrt(jnp.where(pos, deg, 1.0)), 0.0)
    dis_ref[...] = dis
    um_ref[...] = dis[:, None] * u0_ref[...]


def _topk_mask(score2d):
    ibits = lax.bitcast_convert_type(score2d, jnp.int32)
    key = jnp.where(ibits >= 0, ibits, ibits ^ jnp.int32(0x7FFFFFFF))
    u = key ^ jnp.int32(-2147483648)
    hi = lax.shift_right_logical(u, 16)
    lo = u & jnp.int32(0xFFFF)

    def _find(vals, base_cnt, extra):
        def body(j, t):
            cand = t + lax.shift_left(jnp.int32(1), jnp.int32(15) - j)
            cnt = base_cnt + jnp.sum(jnp.where(extra & (vals >= cand), 1, 0))
            return jnp.where(cnt >= K, cand, t)
        return lax.fori_loop(0, 16, body, jnp.int32(0))

    ones = jnp.ones_like(hi, dtype=jnp.bool_)
    th = _find(hi, jnp.int32(0), ones)
    cnt_hi_gt = jnp.sum(jnp.where(hi > th, 1, 0))
    eq_hi = hi == th
    tl = _find(lo, cnt_hi_gt, eq_hi)
    gt = (hi > th) | (eq_hi & (lo > tl))
    eq = eq_hi & (lo == tl)
    r = K - jnp.sum(jnp.where(gt, 1, 0))

    eqf = jnp.where(eq, 1.0, 0.0)
    col = lax.broadcasted_iota(jnp.int32, (128, 128), 0)
    row = lax.broadcasted_iota(jnp.int32, (128, 128), 1)
    ut = jnp.where(col <= row, 1.0, 0.0)
    within = jnp.dot(eqf, ut, preferred_element_type=jnp.float32)
    rowtot = within[:, 127:128]
    a = lax.broadcasted_iota(jnp.int32, (80, 80), 0)
    b = lax.broadcasted_iota(jnp.int32, (80, 80), 1)
    sl = jnp.where(b < a, 1.0, 0.0)
    before = jnp.dot(sl, rowtot, preferred_element_type=jnp.float32)
    cum = before + within
    sel_eq = eq & (cum <= r.astype(jnp.float32))
    return jnp.where(gt | sel_eq, 1.0, 0.0)


def _k_add(p_ref, o_ref):
    o_ref[...] = p_ref[0] + p_ref[1]


def _k_score(um_ref, t1_ref, t2_ref, t3_ref, dis_ref, s2_ref, b1_ref, p_ref,
             beta_ref, pw_ref, h_ref, score_ref):
    pw = pw_ref[...]
    dis = dis_ref[...]
    m = (pw[0] * um_ref[...]
         + pw[1] * t1_ref[...]
         + pw[2] * t2_ref[...]
         + pw[3] * t3_ref[...])
    h = jnp.maximum(dis[:, None] * m + b1_ref[...][None, :], 0.0)
    rowid = lax.broadcasted_iota(jnp.int32, (NP, 1), 0)[:, 0]
    valid = rowid < N
    h = jnp.where(valid[:, None], h, 0.0)
    score1 = jnp.dot(h, p_ref[...][:, None],
                     preferred_element_type=jnp.float32)[:, 0]
    beta = beta_ref[...]
    score = jnp.tanh(beta[0] * score1 + beta[1] * (dis * s2_ref[...]))
    score = jnp.where(valid, score, -3.0e38)
    h_ref[...] = h
    score_ref[...] = score


def _k_mask(score2d_ref, mask2d_ref):
    mask2d_ref[...] = _topk_mask(score2d_ref[...])


def _k_xw(h_ref, score_ref, mask_ref, dis_ref, w2_ref, xw_ref, v0_ref):
    maskf = mask_ref[...]
    score = score_ref[...]
    xp = maskf[:, None] * h_ref[...] * score[:, None]
    xw_ref[...] = jnp.dot(xp, w2_ref[...], preferred_element_type=jnp.float32)
    v0_ref[...] = dis_ref[...] * maskf


def _k_deg2(mch_ref, dis_ref, mask_ref, xw_ref, d2_ref, z_ref, u2_ref):
    maskf = mask_ref[...]
    dis = dis_ref[...]
    deg2 = maskf * (dis * mch_ref[...]) + maskf
    pos = deg2 > 0.0
    d2 = jnp.where(pos, lax.rsqrt(jnp.where(pos, deg2, 1.0)), 0.0) * maskf
    z = d2[:, None] * xw_ref[...]
    d2_ref[...] = d2
    z_ref[...] = z
    u2_ref[...] = dis[:, None] * z


def _k_final(u2_ref, s1_ref, s2_ref, s3_ref, dis_ref, mask_ref, d2_ref,
             z_ref, b2_ref, w3_ref, b3_ref, pw_ref, out_ref):
    pw = pw_ref[...]
    m2 = (pw[0] * u2_ref[...]
          + pw[1] * s1_ref[...]
          + pw[2] * s2_ref[...]
          + pw[3] * s3_ref[...])
    dis = dis_ref[...]
    maskf = mask_ref[...]
    d2 = d2_ref[...]
    a2z = maskf[:, None] * (dis[:, None] * m2) + z_ref[...]
    h2 = maskf[:, None] * jnp.maximum(d2[:, None] * a2z + b2_ref[...][None, :], 0.0)
    g = jnp.sum(h2, axis=0, keepdims=True)
    logits = jnp.dot(g, w3_ref[...], preferred_element_type=jnp.float32) \
        + b3_ref[...][None, :]
    mx = jnp.max(logits, axis=-1, keepdims=True)
    sh = logits - mx
    out_ref[...] = sh - jnp.log(jnp.sum(jnp.exp(sh), axis=-1, keepdims=True))


def kernel(x, edge_index, edge_weight, batch, W1, b1, pan_w, p, beta, W2, b2,
           W3, b3):
    src = edge_index[0]
    dst = edge_index[1]
    ew = edge_weight
    src2 = src.reshape(E // CH, CH)
    dst2 = dst.reshape(E // CH, CH)
    ew2 = ew.reshape(E // CH, CH)
    xp = jnp.pad(x, ((0, NP - N), (0, 0)))
    zer_nh = jnp.zeros((NP, H), jnp.float32)
    zer_n = jnp.zeros((NP,), jnp.float32)
    ones_n = jnp.pad(jnp.ones((N,), jnp.float32), (0, NP - N))
    pw16 = jnp.broadcast_to(pan_w[:, None], (L + 1, 16))

    u0 = _tc_call(_k_xw1, jax.ShapeDtypeStruct((NP, H), jnp.float32), xp, W1)

    deg = _get_vchain()(ones_n, src, dst, ew, zer_n, pw16)
    dis, um = _tc_call(
        _k_dis_um,
        (jax.ShapeDtypeStruct((NP,), jnp.float32),
         jax.ShapeDtypeStruct((NP, H), jnp.float32)),
        deg, u0)

    s2raw = _get_vchain()(dis, dst, src, ew, zer_n, pw16)

    nh = jax.ShapeDtypeStruct((NP, H), jnp.float32)
    t1 = _tc_call(_k_add, nh, _get_spmv()(um, src2, dst2, ew2, zer_nh))
    t2 = _tc_call(_k_add, nh, _get_spmv()(t1, src2, dst2, ew2, zer_nh))
    t3 = _tc_call(_k_add, nh, _get_spmv()(t2, src2, dst2, ew2, zer_nh))

    h, score = _tc_call(
        _k_score,
        (jax.ShapeDtypeStruct((NP, H), jnp.float32),
         jax.ShapeDtypeStruct((NP,), jnp.float32)),
        um, t1, t2, t3, dis, s2raw, b1, p, beta, pan_w)
    mask2d = _tc_call(_k_mask, jax.ShapeDtypeStruct((80, 128), jnp.float32),
                      score.reshape(80, 128))
    maskf = mask2d.reshape(NP)
    xw, v0 = _tc_call(
        _k_xw,
        (jax.ShapeDtypeStruct((NP, H), jnp.float32),
         jax.ShapeDtypeStruct((NP,), jnp.float32)),
        h, score, maskf, dis, W2)

    mch = _get_vchain()(v0, src, dst, ew, zer_n, pw16)
    d2, z, u2 = _tc_call(
        _k_deg2,
        (jax.ShapeDtypeStruct((NP,), jnp.float32),
         jax.ShapeDtypeStruct((NP, H), jnp.float32),
         jax.ShapeDtypeStruct((NP, H), jnp.float32)),
        mch, dis, maskf, xw)

    c1 = _tc_call(_k_add, nh, _get_spmv()(u2, src2, dst2, ew2, zer_nh))
    c2 = _tc_call(_k_add, nh, _get_spmv()(c1, src2, dst2, ew2, zer_nh))
    c3 = _tc_call(_k_add, nh, _get_spmv()(c2, src2, dst2, ew2, zer_nh))

    out = _tc_call(
        _k_final, jax.ShapeDtypeStruct((1, C), jnp.float32),
        u2, c1, c2, c3, dis, maskf, d2, z, b2, W3, b3, pan_w)
    return out


# speedup vs baseline: 1.6775x; 1.4707x over previous
import functools
import math

import jax
import jax.numpy as jnp
from jax import lax
from jax.experimental import pallas as pl
from jax.experimental.pallas import tpu as pltpu
from jax.experimental.pallas import tpu_sc as plsc

N = 10000
E = 320000
F_IN = 128
H = 64
C = 2
L = 3
K = int(math.ceil(0.25 * N))

NP = 10240
NC = 2
NS = 16
NW = NC * NS
RPT = NP // NS
EPW = E // NW
EPT = E // NS
CH = 400
NCH = EPW // CH

@functools.cache
def _get_spmv():
    mesh = plsc.VectorSubcoreMesh(core_axis_name="c", subcore_axis_name="s")
    return functools.partial(
        pl.kernel,
        mesh=mesh,
        out_type=jax.ShapeDtypeStruct((NC, NP, H), jnp.float32),
        scratch_types=[
            pltpu.VMEM_SHARED((NP, H), jnp.float32),
            pltpu.VMEM((CH, H), jnp.float32),
            pltpu.VMEM((CH, H), jnp.float32),
            pltpu.VMEM((NCH, CH), jnp.int32),
            pltpu.VMEM((NCH, CH), jnp.int32),
            pltpu.VMEM((NCH, CH), jnp.float32),
            pltpu.SemaphoreType.DMA,
            pltpu.SemaphoreType.DMA,
            pltpu.SemaphoreType.DMA,
            pltpu.SemaphoreType.DMA,
        ],
        compiler_params=pltpu.CompilerParams(needs_layout_passes=False, use_tc_tiling_on_sc=False),
    )(_sc_spmv_body)


def _sc_spmv_body(ya, src_h, dst_h, ew_h, zer_h, out, acc_s, gbA, gbB,
                  srcb, dstb, ewb, gsA, gsB, ssA, ssB):
    cid = lax.axis_index("c")
    tid = lax.axis_index("s")
    wid = cid * NS + tid
    rbase = tid * RPT
    cbase = wid * NCH

    pltpu.sync_copy(zer_h.at[pl.ds(rbase, RPT)], acc_s.at[pl.ds(rbase, RPT)])
    pltpu.sync_copy(src_h.at[pl.ds(cbase, NCH)], srcb)
    pltpu.sync_copy(dst_h.at[pl.ds(cbase, NCH)], dstb)
    pltpu.sync_copy(ew_h.at[pl.ds(cbase, NCH)], ewb)
    plsc.subcore_barrier()

    def _bufs(j):
        return (gbA, gsA, ssA) if j % 2 == 0 else (gbB, gsB, ssB)

    def _g_start(j):
        gb, gs, _ = _bufs(j)
        pltpu.async_copy(ya.at[srcb.at[j]], gb, gs)

    def _g_wait(j):
        gb, gs, _ = _bufs(j)
        pltpu.make_async_copy(ya.at[srcb.at[j]], gb, gs).wait()

    def _s_start(j):
        gb, _, ss = _bufs(j)
        pltpu.async_copy(gb, acc_s.at[dstb.at[j]], ss, add=True)

    def _s_wait(j):
        gb, _, ss = _bufs(j)
        pltpu.make_async_copy(gb, acc_s.at[dstb.at[j]], ss).wait()

    def _scale_chunk(j):
        gb, _, _ = _bufs(j)

        def _scale(g, _):
            ewv = ewb[j, pl.ds(g * 16, 16)]
            for b in range(16):
                w = jnp.full((16,), ewv[b])
                e = g * 16 + b
                for q in range(H // 16):
                    gb[e, pl.ds(q * 16, 16)] = gb[e, pl.ds(q * 16, 16)] * w
            return 0
        lax.fori_loop(0, CH // 16, _scale, 0)

    _g_start(0)
    for j in range(NCH):
        if j + 1 < NCH:
            if j + 1 >= 2:
                _s_wait(j - 1)
            _g_start(j + 1)
        _g_wait(j)
        _scale_chunk(j)
        _s_start(j)
    _s_wait(NCH - 2)
    _s_wait(NCH - 1)
    plsc.subcore_barrier()
    pltpu.sync_copy(acc_s.at[pl.ds(rbase, RPT)], out.at[cid, pl.ds(rbase, RPT)])


@functools.cache
def _get_vchain():
    mesh = plsc.VectorSubcoreMesh(core_axis_name="c", subcore_axis_name="s")
    return functools.partial(
        pl.kernel,
        mesh=mesh,
        out_type=jax.ShapeDtypeStruct((NP,), jnp.float32),
        scratch_types=[
            pltpu.VMEM_SHARED((NP,), jnp.float32),
            pltpu.VMEM((NP,), jnp.float32),
            pltpu.VMEM((RPT,), jnp.float32),
            pltpu.VMEM((EPT,), jnp.int32),
            pltpu.VMEM((EPT,), jnp.int32),
            pltpu.VMEM((EPT,), jnp.float32),
            pltpu.VMEM((EPT,), jnp.float32),
            pltpu.VMEM((L + 1, 16), jnp.float32),
        ],
        compiler_params=pltpu.CompilerParams(needs_layout_passes=False, use_tc_tiling_on_sc=False),
    )(_sc_vchain_body)


def _sc_vchain_body(y0_h, src_h, dst_h, ew_h, zer_h, pw_h, out, acc_s, ytile,
                    res_t, srcb, dstb, ewb, vbuf, wbuf):
    cid = lax.axis_index("c")
    tid = lax.axis_index("s")

    @pl.when(cid == 0)
    def _():
        rbase = tid * RPT
        ebase = tid * EPT
        pltpu.sync_copy(src_h.at[pl.ds(ebase, EPT)], srcb)
        pltpu.sync_copy(dst_h.at[pl.ds(ebase, EPT)], dstb)
        pltpu.sync_copy(ew_h.at[pl.ds(ebase, EPT)], ewb)
        pltpu.sync_copy(pw_h, wbuf)
        pltpu.sync_copy(y0_h, ytile)
        pltpu.sync_copy(zer_h.at[pl.ds(rbase, RPT)], acc_s.at[pl.ds(rbase, RPT)])

        w0 = wbuf[0]

        def _init(i, _):
            res_t[pl.ds(i * 16, 16)] = ytile[pl.ds(rbase + i * 16, 16)] * w0
            return 0
        lax.fori_loop(0, RPT // 16, _init, 0)
        plsc.subcore_barrier()

        for k in range(1, L + 1):
            def _edges(i, _):
                sv = srcb[pl.ds(i * 16, 16)]
                vals = plsc.load_gather(ytile, [sv])
                vbuf[pl.ds(i * 16, 16)] = vals * ewb[pl.ds(i * 16, 16)]
                return 0
            lax.fori_loop(0, EPT // 16, _edges, 0)
            pltpu.sync_copy(vbuf, acc_s.at[dstb], add=True)
            plsc.subcore_barrier()
            pltpu.sync_copy(acc_s, ytile)
            plsc.subcore_barrier()
            if k < L:
                pltpu.sync_copy(zer_h.at[pl.ds(rbase, RPT)],
                                acc_s.at[pl.ds(rbase, RPT)])
            wk = wbuf[k]

            def _accum(i, _):
                res_t[pl.ds(i * 16, 16)] = (res_t[pl.ds(i * 16, 16)]
                                            + ytile[pl.ds(rbase + i * 16, 16)] * wk)
                return 0
            lax.fori_loop(0, RPT // 16, _accum, 0)
            plsc.subcore_barrier()

        pltpu.sync_copy(res_t, out.at[pl.ds(rbase, RPT)])


def _tc_call(body, out_shapes, *args):
    return pl.pallas_call(
        body,
        out_shape=out_shapes,
    )(*args)


def _k_xw1(x_ref, w_ref, o_ref):
    o_ref[...] = jnp.dot(x_ref[...], w_ref[...],
                         preferred_element_type=jnp.float32)


def _k_dis_um(deg_ref, u0_ref, dis_ref, um_ref):
    deg = deg_ref[...]
    pos = deg > 0.0
    dis = jnp.where(pos, lax.rsqrt(jnp.where(pos, deg, 1.0)), 0.0)
    dis_ref[...] = dis
    um_ref[...] = dis[:, None] * u0_ref[...]


def _topk_mask(score2d):
    ibits = lax.bitcast_convert_type(score2d, jnp.int32)
    key = jnp.where(ibits >= 0, ibits, ibits ^ jnp.int32(0x7FFFFFFF))
    u = key ^ jnp.int32(-2147483648)
    hi = lax.shift_right_logical(u, 16)
    lo = u & jnp.int32(0xFFFF)

    def _find(vals, base_cnt, extra):
        def body(j, t):
            cand = t + lax.shift_left(jnp.int32(1), jnp.int32(15) - j)
            cnt = base_cnt + jnp.sum(jnp.where(extra & (vals >= cand), 1, 0))
            return jnp.where(cnt >= K, cand, t)
        return lax.fori_loop(0, 16, body, jnp.int32(0))

    ones = jnp.ones_like(hi, dtype=jnp.bool_)
    th = _find(hi, jnp.int32(0), ones)
    cnt_hi_gt = jnp.sum(jnp.where(hi > th, 1, 0))
    eq_hi = hi == th
    tl = _find(lo, cnt_hi_gt, eq_hi)
    gt = (hi > th) | (eq_hi & (lo > tl))
    eq = eq_hi & (lo == tl)
    r = K - jnp.sum(jnp.where(gt, 1, 0))

    eqf = jnp.where(eq, 1.0, 0.0)
    col = lax.broadcasted_iota(jnp.int32, (128, 128), 0)
    row = lax.broadcasted_iota(jnp.int32, (128, 128), 1)
    ut = jnp.where(col <= row, 1.0, 0.0)
    within = jnp.dot(eqf, ut, preferred_element_type=jnp.float32)
    rowtot = within[:, 127:128]
    a = lax.broadcasted_iota(jnp.int32, (80, 80), 0)
    b = lax.broadcasted_iota(jnp.int32, (80, 80), 1)
    sl = jnp.where(b < a, 1.0, 0.0)
    before = jnp.dot(sl, rowtot, preferred_element_type=jnp.float32)
    cum = before + within
    sel_eq = eq & (cum <= r.astype(jnp.float32))
    return jnp.where(gt | sel_eq, 1.0, 0.0)


def _k_add(p_ref, o_ref):
    o_ref[...] = p_ref[0] + p_ref[1]


def _k_score(um_ref, t1_ref, t2_ref, t3_ref, dis_ref, s2_ref, b1_ref, p_ref,
             beta_ref, pw_ref, h_ref, score_ref):
    pw = pw_ref[...]
    dis = dis_ref[...]
    m = (pw[0] * um_ref[...]
         + pw[1] * t1_ref[...]
         + pw[2] * t2_ref[...]
         + pw[3] * t3_ref[...])
    h = jnp.maximum(dis[:, None] * m + b1_ref[...][None, :], 0.0)
    rowid = lax.broadcasted_iota(jnp.int32, (NP, 1), 0)[:, 0]
    valid = rowid < N
    h = jnp.where(valid[:, None], h, 0.0)
    score1 = jnp.dot(h, p_ref[...][:, None],
                     preferred_element_type=jnp.float32)[:, 0]
    beta = beta_ref[...]
    score = jnp.tanh(beta[0] * score1 + beta[1] * (dis * s2_ref[...]))
    score = jnp.where(valid, score, -3.0e38)
    h_ref[...] = h
    score_ref[...] = score


def _k_mask(score2d_ref, mask2d_ref):
    mask2d_ref[...] = _topk_mask(score2d_ref[...])


def _k_xw(h_ref, score_ref, mask_ref, dis_ref, w2_ref, xw_ref, v0_ref):
    maskf = mask_ref[...]
    score = score_ref[...]
    xp = maskf[:, None] * h_ref[...] * score[:, None]
    xw_ref[...] = jnp.dot(xp, w2_ref[...], preferred_element_type=jnp.float32)
    v0_ref[...] = dis_ref[...] * maskf


def _k_deg2(mch_ref, dis_ref, mask_ref, xw_ref, d2_ref, z_ref, u2_ref):
    maskf = mask_ref[...]
    dis = dis_ref[...]
    deg2 = maskf * (dis * mch_ref[...]) + maskf
    pos = deg2 > 0.0
    d2 = jnp.where(pos, lax.rsqrt(jnp.where(pos, deg2, 1.0)), 0.0) * maskf
    z = d2[:, None] * xw_ref[...]
    d2_ref[...] = d2
    z_ref[...] = z
    u2_ref[...] = dis[:, None] * z


def _k_final(u2_ref, s1_ref, s2_ref, s3_ref, dis_ref, mask_ref, d2_ref,
             z_ref, b2_ref, w3_ref, b3_ref, pw_ref, out_ref):
    pw = pw_ref[...]
    m2 = (pw[0] * u2_ref[...]
          + pw[1] * s1_ref[...]
          + pw[2] * s2_ref[...]
          + pw[3] * s3_ref[...])
    dis = dis_ref[...]
    maskf = mask_ref[...]
    d2 = d2_ref[...]
    a2z = maskf[:, None] * (dis[:, None] * m2) + z_ref[...]
    h2 = maskf[:, None] * jnp.maximum(d2[:, None] * a2z + b2_ref[...][None, :], 0.0)
    g = jnp.sum(h2, axis=0, keepdims=True)
    logits = jnp.dot(g, w3_ref[...], preferred_element_type=jnp.float32) \
        + b3_ref[...][None, :]
    mx = jnp.max(logits, axis=-1, keepdims=True)
    sh = logits - mx
    out_ref[...] = sh - jnp.log(jnp.sum(jnp.exp(sh), axis=-1, keepdims=True))


def kernel(x, edge_index, edge_weight, batch, W1, b1, pan_w, p, beta, W2, b2,
           W3, b3):
    src = edge_index[0]
    dst = edge_index[1]
    ew = edge_weight
    src2 = src.reshape(E // CH, CH)
    dst2 = dst.reshape(E // CH, CH)
    ew2 = ew.reshape(E // CH, CH)
    xp = jnp.pad(x, ((0, NP - N), (0, 0)))
    zer_nh = jnp.zeros((NP, H), jnp.float32)
    zer_n = jnp.zeros((NP,), jnp.float32)
    ones_n = jnp.pad(jnp.ones((N,), jnp.float32), (0, NP - N))
    pw16 = jnp.broadcast_to(pan_w[:, None], (L + 1, 16))

    u0 = _tc_call(_k_xw1, jax.ShapeDtypeStruct((NP, H), jnp.float32), xp, W1)

    deg = _get_vchain()(ones_n, src, dst, ew, zer_n, pw16)
    dis, um = _tc_call(
        _k_dis_um,
        (jax.ShapeDtypeStruct((NP,), jnp.float32),
         jax.ShapeDtypeStruct((NP, H), jnp.float32)),
        deg, u0)

    s2raw = _get_vchain()(dis, dst, src, ew, zer_n, pw16)

    nh = jax.ShapeDtypeStruct((NP, H), jnp.float32)
    t1 = _tc_call(_k_add, nh, _get_spmv()(um, src2, dst2, ew2, zer_nh))
    t2 = _tc_call(_k_add, nh, _get_spmv()(t1, src2, dst2, ew2, zer_nh))
    t3 = _tc_call(_k_add, nh, _get_spmv()(t2, src2, dst2, ew2, zer_nh))

    h, score = _tc_call(
        _k_score,
        (jax.ShapeDtypeStruct((NP, H), jnp.float32),
         jax.ShapeDtypeStruct((NP,), jnp.float32)),
        um, t1, t2, t3, dis, s2raw, b1, p, beta, pan_w)
    mask2d = _tc_call(_k_mask, jax.ShapeDtypeStruct((80, 128), jnp.float32),
                      score.reshape(80, 128))
    maskf = mask2d.reshape(NP)
    xw, v0 = _tc_call(
        _k_xw,
        (jax.ShapeDtypeStruct((NP, H), jnp.float32),
         jax.ShapeDtypeStruct((NP,), jnp.float32)),
        h, score, maskf, dis, W2)

    mch = _get_vchain()(v0, src, dst, ew, zer_n, pw16)
    d2, z, u2 = _tc_call(
        _k_deg2,
        (jax.ShapeDtypeStruct((NP,), jnp.float32),
         jax.ShapeDtypeStruct((NP, H), jnp.float32),
         jax.ShapeDtypeStruct((NP, H), jnp.float32)),
        mch, dis, maskf, xw)

    c1 = _tc_call(_k_add, nh, _get_spmv()(u2, src2, dst2, ew2, zer_nh))
    c2 = _tc_call(_k_add, nh, _get_spmv()(c1, src2, dst2, ew2, zer_nh))
    c3 = _tc_call(_k_add, nh, _get_spmv()(c2, src2, dst2, ew2, zer_nh))

    out = _tc_call(
        _k_final, jax.ShapeDtypeStruct((1, C), jnp.float32),
        u2, c1, c2, c3, dis, maskf, d2, z, b2, W3, b3, pan_w)
    return out


# speedup vs baseline: 1.6778x; 1.0001x over previous
import functools
import math

import jax
import jax.numpy as jnp
from jax import lax
from jax.experimental import pallas as pl
from jax.experimental.pallas import tpu as pltpu
from jax.experimental.pallas import tpu_sc as plsc

N = 10000
E = 320000
F_IN = 128
H = 64
C = 2
L = 3
K = int(math.ceil(0.25 * N))

NP = 10240
NC = 2
NS = 16
NW = NC * NS
RPT = NP // NS
EPW = E // NW
EPT = E // NS
CH = 400
NCH = EPW // CH

@functools.cache
def _get_spmv():
    mesh = plsc.VectorSubcoreMesh(core_axis_name="c", subcore_axis_name="s")
    return functools.partial(
        pl.kernel,
        mesh=mesh,
        out_type=jax.ShapeDtypeStruct((NC, NP, H), jnp.float32),
        scratch_types=[
            pltpu.VMEM_SHARED((NP, H), jnp.float32),
            pltpu.VMEM((CH, H), jnp.float32),
            pltpu.VMEM((CH, H), jnp.float32),
            pltpu.VMEM((NCH, CH), jnp.int32),
            pltpu.VMEM((NCH, CH), jnp.int32),
            pltpu.VMEM((NCH, CH), jnp.float32),
            pltpu.SemaphoreType.DMA,
            pltpu.SemaphoreType.DMA,
            pltpu.SemaphoreType.DMA,
            pltpu.SemaphoreType.DMA,
        ],
        compiler_params=pltpu.CompilerParams(needs_layout_passes=False, use_tc_tiling_on_sc=False),
    )(_sc_spmv_body)


def _sc_spmv_body(ya, src_h, dst_h, ew_h, zer_h, out, acc_s, gbA, gbB,
                  srcb, dstb, ewb, gsA, gsB, ssA, ssB):
    cid = lax.axis_index("c")
    tid = lax.axis_index("s")
    wid = cid * NS + tid
    rbase = tid * RPT
    cbase = wid * NCH

    pltpu.sync_copy(zer_h.at[pl.ds(rbase, RPT)], acc_s.at[pl.ds(rbase, RPT)])
    pltpu.sync_copy(src_h.at[pl.ds(cbase, NCH)], srcb)
    pltpu.sync_copy(dst_h.at[pl.ds(cbase, NCH)], dstb)
    pltpu.sync_copy(ew_h.at[pl.ds(cbase, NCH)], ewb)
    plsc.subcore_barrier()

    def _bufs(j):
        return (gbA, gsA, ssA) if j % 2 == 0 else (gbB, gsB, ssB)

    def _g_start(j):
        gb, gs, _ = _bufs(j)
        pltpu.async_copy(ya.at[srcb.at[j]], gb, gs)

    def _g_wait(j):
        gb, gs, _ = _bufs(j)
        pltpu.make_async_copy(ya.at[srcb.at[j]], gb, gs).wait()

    def _s_start(j):
        gb, _, ss = _bufs(j)
        pltpu.async_copy(gb, acc_s.at[dstb.at[j]], ss, add=True)

    def _s_wait(j):
        gb, _, ss = _bufs(j)
        pltpu.make_async_copy(gb, acc_s.at[dstb.at[j]], ss).wait()

    def _scale_chunk(j):
        gb, _, _ = _bufs(j)

        def _scale(g, _):
            ewv = ewb[j, pl.ds(g * 16, 16)]
            for b in range(16):
                w = jnp.full((16,), ewv[b])
                e = g * 16 + b
                for q in range(H // 16):
                    gb[e, pl.ds(q * 16, 16)] = gb[e, pl.ds(q * 16, 16)] * w
            return 0
        lax.fori_loop(0, CH // 16, _scale, 0)

    _g_start(0)
    for j in range(NCH):
        if j + 1 < NCH:
            if j + 1 >= 2:
                _s_wait(j - 1)
            _g_start(j + 1)
        _g_wait(j)
        _scale_chunk(j)
        _s_start(j)
    _s_wait(NCH - 2)
    _s_wait(NCH - 1)
    plsc.subcore_barrier()
    pltpu.sync_copy(acc_s.at[pl.ds(rbase, RPT)], out.at[cid, pl.ds(rbase, RPT)])


@functools.cache
def _get_vchain():
    mesh = plsc.VectorSubcoreMesh(core_axis_name="c", subcore_axis_name="s")
    return functools.partial(
        pl.kernel,
        mesh=mesh,
        out_type=jax.ShapeDtypeStruct((NP,), jnp.float32),
        scratch_types=[
            pltpu.VMEM_SHARED((NP,), jnp.float32),
            pltpu.VMEM((NP,), jnp.float32),
            pltpu.VMEM((RPT,), jnp.float32),
            pltpu.VMEM((EPT,), jnp.int32),
            pltpu.VMEM((EPT,), jnp.int32),
            pltpu.VMEM((EPT,), jnp.float32),
            pltpu.VMEM((EPT,), jnp.float32),
            pltpu.VMEM((L + 1, 16), jnp.float32),
        ],
        compiler_params=pltpu.CompilerParams(needs_layout_passes=False, use_tc_tiling_on_sc=False),
    )(_sc_vchain_body)


def _sc_vchain_body(y0_h, src_h, dst_h, ew_h, zer_h, pw_h, out, acc_s, ytile,
                    res_t, srcb, dstb, ewb, vbuf, wbuf):
    cid = lax.axis_index("c")
    tid = lax.axis_index("s")

    @pl.when(cid == 0)
    def _():
        rbase = tid * RPT
        ebase = tid * EPT
        pltpu.sync_copy(src_h.at[pl.ds(ebase, EPT)], srcb)
        pltpu.sync_copy(dst_h.at[pl.ds(ebase, EPT)], dstb)
        pltpu.sync_copy(ew_h.at[pl.ds(ebase, EPT)], ewb)
        pltpu.sync_copy(pw_h, wbuf)
        pltpu.sync_copy(y0_h, ytile)
        pltpu.sync_copy(zer_h.at[pl.ds(rbase, RPT)], acc_s.at[pl.ds(rbase, RPT)])

        w0 = wbuf[0]

        def _init(i, _):
            res_t[pl.ds(i * 16, 16)] = ytile[pl.ds(rbase + i * 16, 16)] * w0
            return 0
        lax.fori_loop(0, RPT // 16, _init, 0)
        plsc.subcore_barrier()

        for k in range(1, L + 1):
            def _edges(i, _):
                sv = srcb[pl.ds(i * 16, 16)]
                vals = plsc.load_gather(ytile, [sv])
                vbuf[pl.ds(i * 16, 16)] = vals * ewb[pl.ds(i * 16, 16)]
                return 0
            lax.fori_loop(0, EPT // 16, _edges, 0)
            pltpu.sync_copy(vbuf, acc_s.at[dstb], add=True)
            plsc.subcore_barrier()
            pltpu.sync_copy(acc_s, ytile)
            plsc.subcore_barrier()
            if k < L:
                pltpu.sync_copy(zer_h.at[pl.ds(rbase, RPT)],
                                acc_s.at[pl.ds(rbase, RPT)])
            wk = wbuf[k]

            def _accum(i, _):
                res_t[pl.ds(i * 16, 16)] = (res_t[pl.ds(i * 16, 16)]
                                            + ytile[pl.ds(rbase + i * 16, 16)] * wk)
                return 0
            lax.fori_loop(0, RPT // 16, _accum, 0)
            plsc.subcore_barrier()

        pltpu.sync_copy(res_t, out.at[pl.ds(rbase, RPT)])


def _tc_call(body, out_shapes, *args):
    return pl.pallas_call(
        body,
        out_shape=out_shapes,
    )(*args)


def _k_xw1(x_ref, w_ref, o_ref):
    o_ref[...] = jnp.dot(x_ref[...], w_ref[...],
                         preferred_element_type=jnp.float32)


def _k_dis_um(deg_ref, u0_ref, dis_ref, um_ref):
    deg = deg_ref[...]
    pos = deg > 0.0
    dis = jnp.where(pos, lax.rsqrt(jnp.where(pos, deg, 1.0)), 0.0)
    dis_ref[...] = dis
    um_ref[...] = dis[:, None] * u0_ref[...]


def _topk_mask(score2d):
    ibits = lax.bitcast_convert_type(score2d, jnp.int32)
    key = jnp.where(ibits >= 0, ibits, ibits ^ jnp.int32(0x7FFFFFFF))
    u = key ^ jnp.int32(-2147483648)
    hi = lax.shift_right_logical(u, 16)
    lo = u & jnp.int32(0xFFFF)

    def _find(vals, base_cnt, extra):
        def body(j, t):
            cand = t + lax.shift_left(jnp.int32(1), jnp.int32(15) - j)
            cnt = base_cnt + jnp.sum(jnp.where(extra & (vals >= cand), 1, 0))
            return jnp.where(cnt >= K, cand, t)
        return lax.fori_loop(0, 16, body, jnp.int32(0))

    ones = jnp.ones_like(hi, dtype=jnp.bool_)
    th = _find(hi, jnp.int32(0), ones)
    cnt_hi_gt = jnp.sum(jnp.where(hi > th, 1, 0))
    eq_hi = hi == th
    tl = _find(lo, cnt_hi_gt, eq_hi)
    gt = (hi > th) | (eq_hi & (lo > tl))
    eq = eq_hi & (lo == tl)
    r = K - jnp.sum(jnp.where(gt, 1, 0))

    eqf = jnp.where(eq, 1.0, 0.0)
    col = lax.broadcasted_iota(jnp.int32, (128, 128), 0)
    row = lax.broadcasted_iota(jnp.int32, (128, 128), 1)
    ut = jnp.where(col <= row, 1.0, 0.0)
    within = jnp.dot(eqf, ut, preferred_element_type=jnp.float32)
    rowtot = within[:, 127:128]
    a = lax.broadcasted_iota(jnp.int32, (80, 80), 0)
    b = lax.broadcasted_iota(jnp.int32, (80, 80), 1)
    sl = jnp.where(b < a, 1.0, 0.0)
    before = jnp.dot(sl, rowtot, preferred_element_type=jnp.float32)
    cum = before + within
    sel_eq = eq & (cum <= r.astype(jnp.float32))
    return jnp.where(gt | sel_eq, 1.0, 0.0)


def _k_add(p_ref, o_ref):
    o_ref[...] = p_ref[0] + p_ref[1]


def _k_score(um_ref, t1_ref, t2_ref, t3_ref, dis_ref, s2_ref, b1_ref, p_ref,
             beta_ref, pw_ref, h_ref, score_ref):
    pw = pw_ref[...]
    dis = dis_ref[...]
    m = (pw[0] * um_ref[...]
         + pw[1] * t1_ref[...]
         + pw[2] * t2_ref[...]
         + pw[3] * t3_ref[...])
    h = jnp.maximum(dis[:, None] * m + b1_ref[...][None, :], 0.0)
    rowid = lax.broadcasted_iota(jnp.int32, (NP, 1), 0)[:, 0]
    valid = rowid < N
    h = jnp.where(valid[:, None], h, 0.0)
    score1 = jnp.dot(h, p_ref[...][:, None],
                     preferred_element_type=jnp.float32)[:, 0]
    beta = beta_ref[...]
    score = jnp.tanh(beta[0] * score1 + beta[1] * (dis * s2_ref[...]))
    score_ref[...] = jnp.where(valid, score, -3.0e38)
    h_ref[...] = h


def _k_mask(score2d_ref, mask2d_ref):
    mask2d_ref[...] = _topk_mask(score2d_ref[...])


def _k_xw(h_ref, score_ref, mask_ref, dis_ref, w2_ref, xw_ref, v0_ref):
    maskf = mask_ref[...]
    score = score_ref[...]
    xp = maskf[:, None] * h_ref[...] * score[:, None]
    xw_ref[...] = jnp.dot(xp, w2_ref[...], preferred_element_type=jnp.float32)
    v0_ref[...] = dis_ref[...] * maskf


def _k_deg2(mch_ref, dis_ref, mask_ref, xw_ref, d2_ref, z_ref, u2_ref):
    maskf = mask_ref[...]
    dis = dis_ref[...]
    deg2 = maskf * (dis * mch_ref[...]) + maskf
    pos = deg2 > 0.0
    d2 = jnp.where(pos, lax.rsqrt(jnp.where(pos, deg2, 1.0)), 0.0) * maskf
    z = d2[:, None] * xw_ref[...]
    d2_ref[...] = d2
    z_ref[...] = z
    u2_ref[...] = dis[:, None] * z


def _k_final(u2_ref, s1_ref, s2_ref, s3_ref, dis_ref, mask_ref, d2_ref,
             z_ref, b2_ref, w3_ref, b3_ref, pw_ref, out_ref):
    pw = pw_ref[...]
    m2 = (pw[0] * u2_ref[...]
          + pw[1] * s1_ref[...]
          + pw[2] * s2_ref[...]
          + pw[3] * s3_ref[...])
    dis = dis_ref[...]
    maskf = mask_ref[...]
    d2 = d2_ref[...]
    a2z = maskf[:, None] * (dis[:, None] * m2) + z_ref[...]
    h2 = maskf[:, None] * jnp.maximum(d2[:, None] * a2z + b2_ref[...][None, :], 0.0)
    g = jnp.sum(h2, axis=0, keepdims=True)
    logits = jnp.dot(g, w3_ref[...], preferred_element_type=jnp.float32) \
        + b3_ref[...][None, :]
    mx = jnp.max(logits, axis=-1, keepdims=True)
    sh = logits - mx
    out_ref[...] = sh - jnp.log(jnp.sum(jnp.exp(sh), axis=-1, keepdims=True))


def kernel(x, edge_index, edge_weight, batch, W1, b1, pan_w, p, beta, W2, b2,
           W3, b3):
    src = edge_index[0]
    dst = edge_index[1]
    ew = edge_weight
    src2 = src.reshape(E // CH, CH)
    dst2 = dst.reshape(E // CH, CH)
    ew2 = ew.reshape(E // CH, CH)
    xp = jnp.pad(x, ((0, NP - N), (0, 0)))
    zer_nh = jnp.zeros((NP, H), jnp.float32)
    zer_n = jnp.zeros((NP,), jnp.float32)
    ones_n = jnp.pad(jnp.ones((N,), jnp.float32), (0, NP - N))
    pw16 = jnp.broadcast_to(pan_w[:, None], (L + 1, 16))

    u0 = _tc_call(_k_xw1, jax.ShapeDtypeStruct((NP, H), jnp.float32), xp, W1)

    deg = _get_vchain()(ones_n, src, dst, ew, zer_n, pw16)
    dis, um = _tc_call(
        _k_dis_um,
        (jax.ShapeDtypeStruct((NP,), jnp.float32),
         jax.ShapeDtypeStruct((NP, H), jnp.float32)),
        deg, u0)

    s2raw = _get_vchain()(dis, dst, src, ew, zer_n, pw16)

    nh = jax.ShapeDtypeStruct((NP, H), jnp.float32)
    t1 = _tc_call(_k_add, nh, _get_spmv()(um, src2, dst2, ew2, zer_nh))
    t2 = _tc_call(_k_add, nh, _get_spmv()(t1, src2, dst2, ew2, zer_nh))
    t3 = _tc_call(_k_add, nh, _get_spmv()(t2, src2, dst2, ew2, zer_nh))

    h, score = _tc_call(
        _k_score,
        (jax.ShapeDtypeStruct((NP, H), jnp.float32),
         jax.ShapeDtypeStruct((NP,), jnp.float32)),
        um, t1, t2, t3, dis, s2raw, b1, p, beta, pan_w)
    mask2d = _tc_call(_k_mask, jax.ShapeDtypeStruct((80, 128), jnp.float32),
                      score.reshape(80, 128))
    maskf = mask2d.reshape(NP)
    xw, v0 = _tc_call(
        _k_xw,
        (jax.ShapeDtypeStruct((NP, H), jnp.float32),
         jax.ShapeDtypeStruct((NP,), jnp.float32)),
        h, score, maskf, dis, W2)

    mch = _get_vchain()(v0, src, dst, ew, zer_n, pw16)
    d2, z, u2 = _tc_call(
        _k_deg2,
        (jax.ShapeDtypeStruct((NP,), jnp.float32),
         jax.ShapeDtypeStruct((NP, H), jnp.float32),
         jax.ShapeDtypeStruct((NP, H), jnp.float32)),
        mch, dis, maskf, xw)

    c1 = _tc_call(_k_add, nh, _get_spmv()(u2, src2, dst2, ew2, zer_nh))
    c2 = _tc_call(_k_add, nh, _get_spmv()(c1, src2, dst2, ew2, zer_nh))
    c3 = _tc_call(_k_add, nh, _get_spmv()(c2, src2, dst2, ew2, zer_nh))

    out = _tc_call(
        _k_final, jax.ShapeDtypeStruct((1, C), jnp.float32),
        u2, c1, c2, c3, dis, maskf, d2, z, b2, W3, b3, pan_w)
    return out
